# baseline (speedup 1.0000x reference)
import functools

import jax
import jax.numpy as jnp
from jax import lax
from jax.experimental import pallas as pl
from jax.experimental.pallas import tpu as pltpu
from jax.experimental.pallas import tpu_sc as plsc

_NH, _NP = 4, 8
_H = _W = 180
_C = 256
_NQ = _H * _W
_D = _C // _NH
_TQ = 400
_NQCH = 4
_NCH = _NQ // _NQCH
_NWORK = 32
_NB = 3
_NK = _NB * (((_NCH + _NWORK - 1) // _NWORK + _NB - 1) // _NB)


def _prep_kernel(q_ref, qn_ref, wv_ref, bv_ref, wox_ref, box_ref, woy_ref,
                 boy_ref, wa_ref, ba_ref, gsum_ref, t2_ref, idx_ref, met_ref):
    q = q_ref[...].reshape(_TQ, _C)
    v = jnp.dot(q, wv_ref[...], preferred_element_type=jnp.float32) + bv_ref[...]
    vplus = jnp.dot(qn_ref[...].reshape(8, _C), wv_ref[...],
                    preferred_element_type=jnp.float32) + bv_ref[...]

    ths = []
    for h in range(_NH):
        vh = v[:, h * _D:(h + 1) * _D]
        vh1 = jnp.concatenate([vh[1:], vplus[0:1, h * _D:(h + 1) * _D]],
                              axis=0)
        ths.append(jnp.concatenate([vh, vh1], axis=1))
    t2_ref[...] = jnp.stack(ths, axis=0)

    t = jnp.dot(q, wa_ref[...], preferred_element_type=jnp.float32) + ba_ref[...]
    m = jnp.max(t, axis=1, keepdims=True)
    e = jnp.exp(t - m)
    den = jnp.dot(e, gsum_ref[...], preferred_element_type=jnp.float32)
    aw = e / den

    ox = jnp.dot(q, wox_ref[...], preferred_element_type=jnp.float32) + box_ref[...]
    oy = jnp.dot(q, woy_ref[...], preferred_element_type=jnp.float32) + boy_ref[...]

    i = pl.program_id(0)
    qid = i * _TQ + lax.broadcasted_iota(jnp.int32, (_TQ, 1), 0)
    colf = (qid % _W).astype(jnp.float32)
    rowf = (qid // _W).astype(jnp.float32)

    x = ((colf + 0.5) / _W + ox / _W) * _W - 0.5
    y = ((rowf + 0.5) / _H + oy / _H) * _H - 0.5
    x0i = jnp.floor(x).astype(jnp.int32)
    y0i = jnp.floor(y).astype(jnp.int32)

    h_lane = lax.broadcasted_iota(jnp.int32, (_TQ, 32), 1) // _NP

    xb = jnp.clip(x0i, 0, _W - 2)
    rb0 = jnp.clip(y0i, 0, _H - 1)
    rb1 = jnp.clip(y0i + 1, 0, _H - 1)
    idx_ref[...] = jnp.concatenate(
        [rb0 * _W + xb + h_lane * _NQ, rb1 * _W + xb + h_lane * _NQ],
        axis=1)
    met_ref[...] = jnp.concatenate([x, y, aw], axis=1)


def _matmul_kernel(x_ref, w_ref, b_ref, o_ref):
    o_ref[...] = (jnp.dot(x_ref[...], w_ref[...],
                          preferred_element_type=jnp.float32)
                  + b_ref[...]).reshape(2 * _TQ, _C // 2)


def _sc_gather(idx_hbm, w_hbm, table_hbm, out_hbm, idx_v, w_v, rows_v, out_v,
               w_s, lsem0, lsem1, lsem2, gsem0, gsem1, gsem2, osem0, osem1,
               osem2):
    cid = lax.axis_index("c")
    sid = lax.axis_index("s")
    wid = sid * 2 + cid
    lsem = (lsem0, lsem1, lsem2)
    gsem = (gsem0, gsem1, gsem2)
    osem = (osem0, osem1, osem2)

    def c_of(k):
        return k * _NWORK + wid

    def load(k, b):
        @pl.when(c_of(k) < _NCH)
        def _():
            base = c_of(k) * _NQCH
            pltpu.async_copy(idx_hbm.at[pl.ds(base, _NQCH)], idx_v.at[b],
                             lsem[b])
            pltpu.async_copy(w_hbm.at[pl.ds(base, _NQCH)], w_v.at[b], lsem[b])

    def wait_load(k, b):
        @pl.when(c_of(k) < _NCH)
        def _():
            pltpu.make_async_copy(idx_hbm.at[pl.ds(0, _NQCH)], idx_v.at[b],
                                  lsem[b]).wait()
            pltpu.make_async_copy(w_hbm.at[pl.ds(0, _NQCH)], w_v.at[b],
                                  lsem[b]).wait()

    def fire(k, b):
        @pl.when(c_of(k) < _NCH)
        def _():
            for j in range(_NQCH):
                pltpu.async_copy(table_hbm.at[idx_v.at[b, j]],
                                 rows_v.at[b, j], gsem[b])

    def wait_fire(k, b):
        @pl.when(c_of(k) < _NCH)
        def _():
            for j in range(_NQCH):
                pltpu.make_async_copy(table_hbm.at[idx_v.at[b, j]],
                                      rows_v.at[b, j], gsem[b]).wait()

    def store(k, b):
        @pl.when(c_of(k) < _NCH)
        def _():
            pltpu.async_copy(out_v.at[b],
                             out_hbm.at[pl.ds(c_of(k) * _NQCH, _NQCH)],
                             osem[b])

    def wait_store(k, b):
        @pl.when((k >= 0) & (c_of(k) < _NCH))
        def _():
            pltpu.make_async_copy(out_v.at[b],
                                  out_hbm.at[pl.ds(0, _NQCH)], osem[b]).wait()

    def compute(k, b):
        @pl.when(c_of(k) < _NCH)
        def _():
            def q_body(j, carry):
                for g in range(2):
                    xg = w_v[b, j, pl.ds(g * 16, 16)]
                    yg = w_v[b, j, pl.ds(32 + g * 16, 16)]
                    ag = w_v[b, j, pl.ds(64 + g * 16, 16)]
                    i0 = idx_v[b, j, pl.ds(g * 16, 16)]
                    i1 = idx_v[b, j, pl.ds(32 + g * 16, 16)]
                    hb = ((lax.iota(jnp.int32, 16) >> 3) + 2 * g) * _NQ
                    r0 = (i0 - hb).astype(jnp.float32)
                    r1 = (i1 - hb).astype(jnp.float32)
                    rb0 = (r0 * (1.0 / _W)).astype(jnp.int32).astype(
                        jnp.float32)
                    rb1 = (r1 * (1.0 / _W)).astype(jnp.int32).astype(
                        jnp.float32)
                    xbf = r0 - rb0 * _W
                    wh0 = jnp.maximum(0.0, 1.0 - jnp.abs(xg - xbf))
                    wh1 = jnp.maximum(0.0, 1.0 - jnp.abs(xg - (xbf + 1.0)))
                    wy0 = (jnp.maximum(0.0, 1.0 - jnp.abs(yg - rb0))
                           * (rb1 - rb0))
                    wy1 = jnp.maximum(0.0, 1.0 - jnp.abs(yg - rb1))
                    w_s[pl.ds(g * 16, 16)] = wh0 * wy0 * ag
                    w_s[pl.ds(32 + g * 16, 16)] = wh0 * wy1 * ag
                    w_s[pl.ds(64 + g * 16, 16)] = wh1 * wy0 * ag
                    w_s[pl.ds(96 + g * 16, 16)] = wh1 * wy1 * ag

                z = jnp.zeros((16,), jnp.float32)
                accs = [z] * 16
                for yc in range(2):
                    for g2 in range(2):
                        l0 = yc * 32 + g2 * 16
                        wv0 = w_s[pl.ds(l0, 16)]
                        wv1 = w_s[pl.ds(64 + l0, 16)]
                        for t in range(16):
                            head = g2 * 2 + t // _NP
                            l = l0 + t
                            a0 = wv0[t]
                            a1 = wv1[t]
                            for s in range(4):
                                accs[head * 4 + s] = (
                                    accs[head * 4 + s]
                                    + a0 * rows_v[b, j, l, pl.ds(s * 16, 16)]
                                    + a1 * rows_v[b, j, l,
                                                  pl.ds(_D + s * 16, 16)])
                for h in range(_NH):
                    for s in range(4):
                        out_v[b, j, pl.ds(h * _D + s * 16, 16)] = accs[h * 4 + s]
                return carry

            lax.fori_loop(0, _NQCH, q_body, 0)

    def phase(k, b):
        b2 = (b + 2) % _NB
        wait_fire(k, b)
        wait_load(k + 2, b2)
        fire(k + 2, b2)
        wait_store(k - _NB, b)
        compute(k, b)
        store(k, b)
        load(k + _NB, b)

    load(0, 0)
    wait_load(0, 0)
    fire(0, 0)
    load(1, 1)
    wait_load(1, 1)
    fire(1, 1)
    load(2, 2)

    def trio_body(i, carry):
        phase(_NB * i, 0)
        phase(_NB * i + 1, 1)
        phase(_NB * i + 2, 2)
        return carry

    lax.fori_loop(0, _NK // _NB, trio_body, 0)
    for t in range(_NB):
        wait_store(_NK - _NB + t, t % _NB)


def kernel(inputs, W_value, b_value, W_off, b_off, W_attn, b_attn, W_out,
           b_out):
    B, L, H, W, C = inputs.shape
    q2d = inputs.reshape(_NQ * 2, _C // 2)

    wof = W_off.reshape(_C, _NH, 1, _NP, 2)
    wox = wof[..., 0].reshape(_C, _NH * _NP)
    woy = wof[..., 1].reshape(_C, _NH * _NP)
    bof = b_off.reshape(_NH, 1, _NP, 2)
    box = bof[..., 0].reshape(1, _NH * _NP)
    boy = bof[..., 1].reshape(1, _NH * _NP)
    g = jnp.repeat(jnp.eye(_NH, dtype=jnp.float32), _NP, axis=0)
    gsum = jnp.repeat(g, _NP, axis=1).reshape(_NH * _NP, _NH * _NP)

    grid = _NQ // _TQ
    nqb = _NQ * 2 // 16
    table, idx, met = pl.pallas_call(
        _prep_kernel,
        grid=(grid,),
        in_specs=[
            pl.BlockSpec((2 * _TQ, _C // 2), lambda i: (i, 0)),
            pl.BlockSpec((16, _C // 2),
                         lambda i: (jnp.minimum((i + 1) * (_TQ * 2 // 16),
                                                nqb - 1), 0)),
            pl.BlockSpec((_C, _C), lambda i: (0, 0)),
            pl.BlockSpec((1, _C), lambda i: (0, 0)),
            pl.BlockSpec((_C, 32), lambda i: (0, 0)),
            pl.BlockSpec((1, 32), lambda i: (0, 0)),
            pl.BlockSpec((_C, 32), lambda i: (0, 0)),
            pl.BlockSpec((1, 32), lambda i: (0, 0)),
            pl.BlockSpec((_C, 32), lambda i: (0, 0)),
            pl.BlockSpec((1, 32), lambda i: (0, 0)),
            pl.BlockSpec((32, 32), lambda i: (0, 0)),
        ],
        out_specs=[
            pl.BlockSpec((_NH, _TQ, 2 * _D), lambda i: (0, i, 0)),
            pl.BlockSpec((_TQ, 64), lambda i: (i, 0)),
            pl.BlockSpec((_TQ, 96), lambda i: (i, 0)),
        ],
        out_shape=[
            jax.ShapeDtypeStruct((_NH, _NQ, 2 * _D), jnp.float32),
            jax.ShapeDtypeStruct((_NQ, 64), jnp.int32),
            jax.ShapeDtypeStruct((_NQ, 96), jnp.float32),
        ],
    )(q2d, q2d, W_value, b_value.reshape(1, _C), wox, box, woy, boy, W_attn,
      b_attn.reshape(1, 32), gsum)

    table = table.reshape(_NH * _NQ, 2 * _D)

    sc_call = functools.partial(
        pl.kernel,
        out_type=jax.ShapeDtypeStruct((_NQ, _C), jnp.float32),
        mesh=plsc.VectorSubcoreMesh(core_axis_name="c", subcore_axis_name="s"),
        scratch_types=[
            pltpu.VMEM((_NB, _NQCH, 64), jnp.int32),
            pltpu.VMEM((_NB, _NQCH, 96), jnp.float32),
            pltpu.VMEM((_NB, _NQCH, 64, 2 * _D), jnp.float32),
            pltpu.VMEM((_NB, _NQCH, _C), jnp.float32),
            pltpu.VMEM((128,), jnp.float32),
        ] + [pltpu.SemaphoreType.DMA] * 9,
    )(_sc_gather)
    acc = sc_call(idx, met, table)

    proj = pl.pallas_call(
        _matmul_kernel,
        grid=(grid,),
        in_specs=[
            pl.BlockSpec((_TQ, _C), lambda i: (i, 0)),
            pl.BlockSpec((_C, _C), lambda i: (0, 0)),
            pl.BlockSpec((1, _C), lambda i: (0, 0)),
        ],
        out_specs=pl.BlockSpec((2 * _TQ, _C // 2), lambda i: (i, 0)),
        out_shape=jax.ShapeDtypeStruct((_NQ * 2, _C // 2), jnp.float32),
    )(acc, W_out, b_out.reshape(1, _C))

    out = proj.reshape(B, L, C, H, W)
    return jnp.transpose(out, (0, 1, 3, 4, 2))


# speedup vs baseline: 1.4802x; 1.4802x over previous
import functools

import jax
import jax.numpy as jnp
from jax import lax
from jax.experimental import pallas as pl
from jax.experimental.pallas import tpu as pltpu
from jax.experimental.pallas import tpu_sc as plsc

_NH, _NP = 4, 8
_H = _W = 180
_C = 256
_NQ = _H * _W
_D = _C // _NH
_TQ = 400
_NQCH = 4
_NCH = _NQ // _NQCH
_NWORK = 32
_NB = 3
_NK = _NB * (((_NCH + _NWORK - 1) // _NWORK + _NB - 1) // _NB)


def _prep_kernel(q_ref, qn_ref, wv_ref, bv_ref, wox_ref, box_ref, woy_ref,
                 boy_ref, wa_ref, ba_ref, gsum_ref, t2_ref, idx_ref, met_ref):
    q = q_ref[...].reshape(_TQ, _C)
    v = jnp.dot(q, wv_ref[...], preferred_element_type=jnp.float32) + bv_ref[...]
    vplus = jnp.dot(qn_ref[...].reshape(_TQ, _C), wv_ref[...],
                    preferred_element_type=jnp.float32) + bv_ref[...]
    vf = jnp.concatenate([v, vplus], axis=0)

    ths = []
    for h in range(_NH):
        vh = vf[:, h * _D:(h + 1) * _D]
        y0r = jnp.concatenate([vh[0:_TQ], vh[1:_TQ + 1]], axis=1)
        y1r = jnp.concatenate([vh[_W:_TQ + _W], vh[_W + 1:_TQ + _W + 1]],
                              axis=1)
        b0 = lax.bitcast_convert_type(y0r, jnp.int32) + 0x8000
        b1 = lax.bitcast_convert_type(y1r, jnp.int32) + 0x8000
        ths.append(((b0 >> 16) & 0xFFFF) | (b1 & ~0xFFFF))
    t2_ref[...] = jnp.stack(ths, axis=1).reshape(_TQ * _NH, 2 * _D)

    t = jnp.dot(q, wa_ref[...], preferred_element_type=jnp.float32) + ba_ref[...]
    m = jnp.max(t, axis=1, keepdims=True)
    e = jnp.exp(t - m)
    den = jnp.dot(e, gsum_ref[...], preferred_element_type=jnp.float32)
    aw = e / den

    ox = jnp.dot(q, wox_ref[...], preferred_element_type=jnp.float32) + box_ref[...]
    oy = jnp.dot(q, woy_ref[...], preferred_element_type=jnp.float32) + boy_ref[...]

    i = pl.program_id(0)
    qid = i * _TQ + lax.broadcasted_iota(jnp.int32, (_TQ, 1), 0)
    colf = (qid % _W).astype(jnp.float32)
    rowf = (qid // _W).astype(jnp.float32)

    x = ((colf + 0.5) / _W + ox / _W) * _W - 0.5
    y = ((rowf + 0.5) / _H + oy / _H) * _H - 0.5
    x0i = jnp.floor(x).astype(jnp.int32)
    y0i = jnp.floor(y).astype(jnp.int32)

    h_lane = lax.broadcasted_iota(jnp.int32, (_TQ, 32), 1) // _NP

    xb = jnp.clip(x0i, 0, _W - 2)
    yq = jnp.clip(y0i, 0, _H - 2)
    idx_ref[...] = (yq * _W + xb) * _NH + h_lane
    met_ref[...] = jnp.concatenate([x, y, aw], axis=1)


def _matmul_kernel(x_ref, w_ref, b_ref, o_ref):
    o_ref[...] = (jnp.dot(x_ref[...], w_ref[...],
                          preferred_element_type=jnp.float32)
                  + b_ref[...]).reshape(2 * _TQ, _C // 2)


def _sc_gather(idx_hbm, w_hbm, table_hbm, out_hbm, idx_v, w_v, rows_v, out_v,
               w_s, lsem0, lsem1, lsem2, gsem0, gsem1, gsem2, osem0, osem1,
               osem2):
    cid = lax.axis_index("c")
    sid = lax.axis_index("s")
    wid = sid * 2 + cid
    lsem = (lsem0, lsem1, lsem2)
    gsem = (gsem0, gsem1, gsem2)
    osem = (osem0, osem1, osem2)

    def c_of(k):
        return k * _NWORK + wid

    def load(k, b):
        @pl.when(c_of(k) < _NCH)
        def _():
            base = c_of(k) * _NQCH
            pltpu.async_copy(idx_hbm.at[pl.ds(base, _NQCH)], idx_v.at[b],
                             lsem[b])
            pltpu.async_copy(w_hbm.at[pl.ds(base, _NQCH)], w_v.at[b], lsem[b])

    def wait_load(k, b):
        @pl.when(c_of(k) < _NCH)
        def _():
            pltpu.make_async_copy(idx_hbm.at[pl.ds(0, _NQCH)], idx_v.at[b],
                                  lsem[b]).wait()
            pltpu.make_async_copy(w_hbm.at[pl.ds(0, _NQCH)], w_v.at[b],
                                  lsem[b]).wait()

    def fire(k, b):
        @pl.when(c_of(k) < _NCH)
        def _():
            for j in range(_NQCH):
                pltpu.async_copy(table_hbm.at[idx_v.at[b, j]],
                                 rows_v.at[b, j], gsem[b])

    def wait_fire(k, b):
        @pl.when(c_of(k) < _NCH)
        def _():
            for j in range(_NQCH):
                pltpu.make_async_copy(table_hbm.at[idx_v.at[b, j]],
                                      rows_v.at[b, j], gsem[b]).wait()

    def store(k, b):
        @pl.when(c_of(k) < _NCH)
        def _():
            pltpu.async_copy(out_v.at[b],
                             out_hbm.at[pl.ds(c_of(k) * _NQCH, _NQCH)],
                             osem[b])

    def wait_store(k, b):
        @pl.when((k >= 0) & (c_of(k) < _NCH))
        def _():
            pltpu.make_async_copy(out_v.at[b],
                                  out_hbm.at[pl.ds(0, _NQCH)], osem[b]).wait()

    def compute(k, b):
        @pl.when(c_of(k) < _NCH)
        def _():
            def q_body(j, carry):
                for g in range(2):
                    xg = w_v[b, j, pl.ds(g * 16, 16)]
                    yg = w_v[b, j, pl.ds(32 + g * 16, 16)]
                    ag = w_v[b, j, pl.ds(64 + g * 16, 16)]
                    iq = idx_v[b, j, pl.ds(g * 16, 16)]
                    rf = (iq >> 2).astype(jnp.float32)
                    yqf = (rf * (1.0 / _W)).astype(jnp.int32).astype(
                        jnp.float32)
                    xbf = rf - yqf * _W
                    wh0 = jnp.maximum(0.0, 1.0 - jnp.abs(xg - xbf))
                    wh1 = jnp.maximum(0.0, 1.0 - jnp.abs(xg - (xbf + 1.0)))
                    wy0 = jnp.maximum(0.0, 1.0 - jnp.abs(yg - yqf))
                    wy1 = jnp.maximum(0.0, 1.0 - jnp.abs(yg - (yqf + 1.0)))
                    w_s[pl.ds(g * 16, 16)] = wh0 * wy0 * ag
                    w_s[pl.ds(32 + g * 16, 16)] = wh1 * wy0 * ag
                    w_s[pl.ds(64 + g * 16, 16)] = wh0 * wy1 * ag
                    w_s[pl.ds(96 + g * 16, 16)] = wh1 * wy1 * ag

                z = jnp.zeros((16,), jnp.float32)
                for g2 in range(2):
                    accs = [z] * 8
                    w00v = w_s[pl.ds(g2 * 16, 16)]
                    w10v = w_s[pl.ds(32 + g2 * 16, 16)]
                    w01v = w_s[pl.ds(64 + g2 * 16, 16)]
                    w11v = w_s[pl.ds(96 + g2 * 16, 16)]
                    for t in range(16):
                        l = g2 * 16 + t
                        hs = (t // _NP) * 4
                        a00 = w00v[t]
                        a10 = w10v[t]
                        a01 = w01v[t]
                        a11 = w11v[t]
                        for s in range(8):
                            v = rows_v[b, j, l, pl.ds(s * 16, 16)]
                            lo = plsc.bitcast(v << 16, jnp.float32)
                            hi = plsc.bitcast(v & ~0xFFFF, jnp.float32)
                            a = a00 if s < 4 else a10
                            c = a01 if s < 4 else a11
                            si = hs + (s % 4)
                            accs[si] = accs[si] + a * lo + c * hi
                    for hh in range(2):
                        for s in range(4):
                            out_v[b, j,
                                  pl.ds((g2 * 2 + hh) * _D + s * 16, 16)] = (
                                accs[hh * 4 + s])
                return carry

            lax.fori_loop(0, _NQCH, q_body, 0)

    def phase(k, b):
        b2 = (b + 2) % _NB
        wait_fire(k, b)
        wait_load(k + 2, b2)
        fire(k + 2, b2)
        wait_store(k - _NB, b)
        compute(k, b)
        store(k, b)
        load(k + _NB, b)

    load(0, 0)
    wait_load(0, 0)
    fire(0, 0)
    load(1, 1)
    wait_load(1, 1)
    fire(1, 1)
    load(2, 2)

    def trio_body(i, carry):
        phase(_NB * i, 0)
        phase(_NB * i + 1, 1)
        phase(_NB * i + 2, 2)
        return carry

    lax.fori_loop(0, _NK // _NB, trio_body, 0)
    for t in range(_NB):
        wait_store(_NK - _NB + t, t % _NB)


def kernel(inputs, W_value, b_value, W_off, b_off, W_attn, b_attn, W_out,
           b_out):
    B, L, H, W, C = inputs.shape
    q2d = inputs.reshape(_NQ * 2, _C // 2)

    wof = W_off.reshape(_C, _NH, 1, _NP, 2)
    wox = wof[..., 0].reshape(_C, _NH * _NP)
    woy = wof[..., 1].reshape(_C, _NH * _NP)
    bof = b_off.reshape(_NH, 1, _NP, 2)
    box = bof[..., 0].reshape(1, _NH * _NP)
    boy = bof[..., 1].reshape(1, _NH * _NP)
    g = jnp.repeat(jnp.eye(_NH, dtype=jnp.float32), _NP, axis=0)
    gsum = jnp.repeat(g, _NP, axis=1).reshape(_NH * _NP, _NH * _NP)

    grid = _NQ // _TQ
    table, idx, met = pl.pallas_call(
        _prep_kernel,
        grid=(grid,),
        in_specs=[
            pl.BlockSpec((2 * _TQ, _C // 2), lambda i: (i, 0)),
            pl.BlockSpec((2 * _TQ, _C // 2),
                         lambda i: (jnp.minimum(i + 1, grid - 1), 0)),
            pl.BlockSpec((_C, _C), lambda i: (0, 0)),
            pl.BlockSpec((1, _C), lambda i: (0, 0)),
            pl.BlockSpec((_C, 32), lambda i: (0, 0)),
            pl.BlockSpec((1, 32), lambda i: (0, 0)),
            pl.BlockSpec((_C, 32), lambda i: (0, 0)),
            pl.BlockSpec((1, 32), lambda i: (0, 0)),
            pl.BlockSpec((_C, 32), lambda i: (0, 0)),
            pl.BlockSpec((1, 32), lambda i: (0, 0)),
            pl.BlockSpec((32, 32), lambda i: (0, 0)),
        ],
        out_specs=[
            pl.BlockSpec((_TQ * _NH, 2 * _D), lambda i: (i, 0)),
            pl.BlockSpec((_TQ, 32), lambda i: (i, 0)),
            pl.BlockSpec((_TQ, 96), lambda i: (i, 0)),
        ],
        out_shape=[
            jax.ShapeDtypeStruct((_NQ * _NH, 2 * _D), jnp.int32),
            jax.ShapeDtypeStruct((_NQ, 32), jnp.int32),
            jax.ShapeDtypeStruct((_NQ, 96), jnp.float32),
        ],
    )(q2d, q2d, W_value, b_value.reshape(1, _C), wox, box, woy, boy, W_attn,
      b_attn.reshape(1, 32), gsum)

    sc_call = functools.partial(
        pl.kernel,
        out_type=jax.ShapeDtypeStruct((_NQ, _C), jnp.float32),
        mesh=plsc.VectorSubcoreMesh(core_axis_name="c", subcore_axis_name="s"),
        compiler_params=pltpu.CompilerParams(needs_layout_passes=False),
        scratch_types=[
            pltpu.VMEM((_NB, _NQCH, 32), jnp.int32),
            pltpu.VMEM((_NB, _NQCH, 96), jnp.float32),
            pltpu.VMEM((_NB, _NQCH, 32, 2 * _D), jnp.int32),
            pltpu.VMEM((_NB, _NQCH, _C), jnp.float32),
            pltpu.VMEM((128,), jnp.float32),
        ] + [pltpu.SemaphoreType.DMA] * 9,
    )(_sc_gather)
    acc = sc_call(idx, met, table)

    proj = pl.pallas_call(
        _matmul_kernel,
        grid=(grid,),
        in_specs=[
            pl.BlockSpec((_TQ, _C), lambda i: (i, 0)),
            pl.BlockSpec((_C, _C), lambda i: (0, 0)),
            pl.BlockSpec((1, _C), lambda i: (0, 0)),
        ],
        out_specs=pl.BlockSpec((2 * _TQ, _C // 2), lambda i: (i, 0)),
        out_shape=jax.ShapeDtypeStruct((_NQ * 2, _C // 2), jnp.float32),
    )(acc, W_out, b_out.reshape(1, _C))

    out = proj.reshape(B, L, C, H, W)
    return jnp.transpose(out, (0, 1, 3, 4, 2))
